# baseline (speedup 1.0000x reference)
import functools

import jax
import jax.numpy as jnp
from jax import lax
from jax.experimental import pallas as pl
from jax.experimental.pallas import tpu as pltpu
from jax.experimental.pallas import tpu_sc as plsc

_BATCH = 16384
_COLS = 512
_PATHWAYS = 64
_GENES = 8

_TC_ROWS = 12288
_SC_ROWS = _BATCH - _TC_ROWS
_TC_BLK = 2048

_NC = 2
_NS = 16
_NW = _NC * _NS
_ROWS_PER_W = _SC_ROWS // _NW
_CHUNK = 64
_NCHUNK = _ROWS_PER_W // _CHUNK


def _sc_body(x_hbm, out_hbm, in0, in1, ou0, ou1, si0, si1, so0, so1):
    wid = lax.axis_index("s") * _NC + lax.axis_index("c")
    base = _TC_ROWS + wid * _ROWS_PER_W
    lane8 = lax.iota(jnp.int32, 16) * 8
    col_idx = [lane8 + (128 * j + g) for j in range(4) for g in range(_GENES)]

    in_bufs = (in0, in1)
    out_bufs = (ou0, ou1)
    in_sems = (si0, si1)
    out_sems = (so0, so1)

    def start_in(ci, b):
        return pltpu.async_copy(
            x_hbm.at[pl.ds(base + ci * _CHUNK, _CHUNK)], in_bufs[b], in_sems[b]
        )

    def one_row(in_v, out_v, r):
        row_idx = jnp.full((16,), r, jnp.int32)
        for j in range(4):
            g = [plsc.load_gather(in_v, [row_idx, col_idx[8 * j + k]])
                 for k in range(_GENES)]
            s = ((g[0] + g[1]) + (g[2] + g[3])) + ((g[4] + g[5]) + (g[6] + g[7]))
            out_v[r, pl.ds(16 * j, 16)] = s * 0.125

    out_copies = {}
    start_in(0, 0)
    for ci in range(_NCHUNK):
        b = ci % 2
        pltpu.make_async_copy(
            x_hbm.at[pl.ds(base + ci * _CHUNK, _CHUNK)], in_bufs[b], in_sems[b]
        ).wait()
        if ci + 1 < _NCHUNK:
            start_in(ci + 1, 1 - b)
        if ci >= 2:
            out_copies[ci - 2].wait()

        @plsc.parallel_loop(0, _CHUNK, step=1, unroll=4)
        def _rows(r):
            one_row(in_bufs[b], out_bufs[b], r)

        out_copies[ci] = pltpu.async_copy(
            out_bufs[b],
            out_hbm.at[pl.ds(wid * _ROWS_PER_W + ci * _CHUNK, _CHUNK)],
            out_sems[b],
        )
    out_copies[_NCHUNK - 2].wait()
    out_copies[_NCHUNK - 1].wait()


def _sc_call(x):
    mesh = plsc.VectorSubcoreMesh(core_axis_name="c", subcore_axis_name="s")
    run = pl.kernel(
        _sc_body,
        out_type=jax.ShapeDtypeStruct((_SC_ROWS, _PATHWAYS), jnp.float32),
        mesh=mesh,
        scratch_types=[
            pltpu.VMEM((_CHUNK, _COLS), jnp.float32),
            pltpu.VMEM((_CHUNK, _COLS), jnp.float32),
            pltpu.VMEM((_CHUNK, _PATHWAYS), jnp.float32),
            pltpu.VMEM((_CHUNK, _PATHWAYS), jnp.float32),
            pltpu.SemaphoreType.DMA,
            pltpu.SemaphoreType.DMA,
            pltpu.SemaphoreType.DMA,
            pltpu.SemaphoreType.DMA,
        ],
        compiler_params=pltpu.CompilerParams(
            use_tc_tiling_on_sc=True, needs_layout_passes=False
        ),
    )
    return run(x)


def _tc_body(x_ref, w_ref, o_ref):
    o_ref[...] = jax.lax.dot_general(
        x_ref[...], w_ref[...], (((1,), (0,)), ((), ())),
        preferred_element_type=jnp.float32,
    )


def _tc_call(x, w):
    return pl.pallas_call(
        _tc_body,
        grid=(_TC_ROWS // _TC_BLK,),
        in_specs=[
            pl.BlockSpec((_TC_BLK, _COLS), lambda i: (i, 0)),
            pl.BlockSpec((_COLS, _PATHWAYS), lambda i: (0, 0)),
        ],
        out_specs=pl.BlockSpec((_TC_BLK, _PATHWAYS), lambda i: (i, 0)),
        out_shape=jax.ShapeDtypeStruct((_TC_ROWS, _PATHWAYS), jnp.float32),
    )(x, w)


_ASM_BLK = 1024
_N_TC_BLKS = _TC_ROWS // _ASM_BLK


def _asm_body(tc_ref, sc_ref, o_ref):
    i = pl.program_id(0)

    @pl.when(i < _N_TC_BLKS)
    def _():
        o_ref[...] = tc_ref[...]

    @pl.when(i >= _N_TC_BLKS)
    def _():
        o_ref[...] = sc_ref[...]


def _asm_call(tc_out, sc_out):
    return pl.pallas_call(
        _asm_body,
        grid=(_BATCH // _ASM_BLK,),
        in_specs=[
            pl.BlockSpec((_ASM_BLK, _PATHWAYS),
                         lambda i: (jnp.minimum(i, _N_TC_BLKS - 1), 0)),
            pl.BlockSpec((_ASM_BLK, _PATHWAYS),
                         lambda i: (jnp.maximum(i - _N_TC_BLKS, 0), 0)),
        ],
        out_specs=pl.BlockSpec((_ASM_BLK, _PATHWAYS), lambda i: (i, 0)),
        out_shape=jax.ShapeDtypeStruct((_BATCH, _PATHWAYS), jnp.float32),
    )(tc_out, sc_out)


@jax.jit
def kernel(geneset_features):
    w = jnp.repeat(jnp.eye(_PATHWAYS, dtype=jnp.float32), _GENES, axis=0) * (
        1.0 / _GENES
    )
    sc_out = _sc_call(geneset_features)
    tc_out = _tc_call(geneset_features, w)
    return _asm_call(tc_out, sc_out)


# speedup vs baseline: 1.1203x; 1.1203x over previous
import functools

import jax
import jax.numpy as jnp
from jax import lax
from jax.experimental import pallas as pl
from jax.experimental.pallas import tpu as pltpu
from jax.experimental.pallas import tpu_sc as plsc

_BATCH = 16384
_COLS = 512
_PATHWAYS = 64
_GENES = 8

_TC_ROWS = 12288
_SC_ROWS = _BATCH - _TC_ROWS
_TC_BLK = 4096

_NC = 2
_NS = 16
_NW = _NC * _NS
_ROWS_PER_W = _SC_ROWS // _NW
_CHUNK = 64
_NCHUNK = _ROWS_PER_W // _CHUNK


def _sc_body(x_hbm, out_hbm, in0, in1, ou0, ou1, si0, si1, so0, so1):
    wid = lax.axis_index("s") * _NC + lax.axis_index("c")
    base = _TC_ROWS + wid * _ROWS_PER_W
    lane8 = lax.iota(jnp.int32, 16) * 8
    col_idx = [lane8 + (128 * j + g) for j in range(4) for g in range(_GENES)]

    in_bufs = (in0, in1)
    out_bufs = (ou0, ou1)
    in_sems = (si0, si1)
    out_sems = (so0, so1)

    def start_in(ci, b):
        return pltpu.async_copy(
            x_hbm.at[pl.ds(base + ci * _CHUNK, _CHUNK)], in_bufs[b], in_sems[b]
        )

    def one_row(in_v, out_v, r):
        row_idx = jnp.full((16,), r, jnp.int32)
        for j in range(4):
            g = [plsc.load_gather(in_v, [row_idx, col_idx[8 * j + k]])
                 for k in range(_GENES)]
            s = ((g[0] + g[1]) + (g[2] + g[3])) + ((g[4] + g[5]) + (g[6] + g[7]))
            out_v[r, pl.ds(16 * j, 16)] = s * 0.125

    out_copies = {}
    start_in(0, 0)
    for ci in range(_NCHUNK):
        b = ci % 2
        pltpu.make_async_copy(
            x_hbm.at[pl.ds(base + ci * _CHUNK, _CHUNK)], in_bufs[b], in_sems[b]
        ).wait()
        if ci + 1 < _NCHUNK:
            start_in(ci + 1, 1 - b)
        if ci >= 2:
            out_copies[ci - 2].wait()

        @plsc.parallel_loop(0, _CHUNK, step=1, unroll=4)
        def _rows(r):
            one_row(in_bufs[b], out_bufs[b], r)

        out_copies[ci] = pltpu.async_copy(
            out_bufs[b],
            out_hbm.at[pl.ds(base + ci * _CHUNK, _CHUNK)],
            out_sems[b],
        )
    out_copies[_NCHUNK - 2].wait()
    out_copies[_NCHUNK - 1].wait()


def _sc_call(x):
    mesh = plsc.VectorSubcoreMesh(core_axis_name="c", subcore_axis_name="s")
    run = pl.kernel(
        _sc_body,
        out_type=jax.ShapeDtypeStruct((_BATCH, _PATHWAYS), jnp.float32),
        mesh=mesh,
        scratch_types=[
            pltpu.VMEM((_CHUNK, _COLS), jnp.float32),
            pltpu.VMEM((_CHUNK, _COLS), jnp.float32),
            pltpu.VMEM((_CHUNK, _PATHWAYS), jnp.float32),
            pltpu.VMEM((_CHUNK, _PATHWAYS), jnp.float32),
            pltpu.SemaphoreType.DMA,
            pltpu.SemaphoreType.DMA,
            pltpu.SemaphoreType.DMA,
            pltpu.SemaphoreType.DMA,
        ],
        compiler_params=pltpu.CompilerParams(
            use_tc_tiling_on_sc=True, needs_layout_passes=False
        ),
    )
    return run(x)


def _tc_body(x_ref, w_ref, sc_ref, o_ref):
    del sc_ref
    o_ref[...] = jax.lax.dot_general(
        x_ref[...], w_ref[...], (((1,), (0,)), ((), ())),
        preferred_element_type=jnp.float32,
    )


def _tc_call(x, w, sc_full):
    return pl.pallas_call(
        _tc_body,
        grid=(_TC_ROWS // _TC_BLK,),
        in_specs=[
            pl.BlockSpec((_TC_BLK, _COLS), lambda i: (i, 0)),
            pl.BlockSpec((_COLS, _PATHWAYS), lambda i: (0, 0)),
            pl.BlockSpec(memory_space=pl.ANY),
        ],
        out_specs=pl.BlockSpec((_TC_BLK, _PATHWAYS), lambda i: (i, 0)),
        out_shape=jax.ShapeDtypeStruct((_BATCH, _PATHWAYS), jnp.float32),
        input_output_aliases={2: 0},
    )(x, w, sc_full)


@jax.jit
def kernel(geneset_features):
    w = jnp.repeat(jnp.eye(_PATHWAYS, dtype=jnp.float32), _GENES, axis=0) * (
        1.0 / _GENES
    )
    sc_full = _sc_call(geneset_features)
    return _tc_call(geneset_features, w, sc_full)
